# Initial kernel scaffold; baseline (speedup 1.0000x reference)
import jax
import jax.numpy as jnp
from jax.experimental import pallas as pl


def kernel(extracted_log_probs, target_lengths, in_idxs, out_idxs, start_idxs, end_idxs):
    raise NotImplementedError("write your pallas kernel here")


# speedup vs baseline: 138.2999x; 138.2999x over previous
import functools
import math

import jax
import jax.numpy as jnp
from jax import lax
from jax.experimental import pallas as pl
from jax.experimental.pallas import tpu as pltpu
from jax.experimental.pallas import tpu_sc as plsc

LOG_EPS = -64.0
EPS = float(math.exp(-64.0))
V, B, T, E, S = 1024, 32, 64, 4096, 4

_NC, _NS, _NL = 2, 16, 16
_E_PER_W = E // _NS
_CH = 128
_NCH = _E_PER_W // _CH
_ZB = 8192
_M_STRIPE = (V * V) // _NS
_OD_STRIPE = V // _NS


def _sc_build_body(in_hbm, out_hbm, m_out, od_out, m_sh, od_sh,
                   obuf, ibuf, fbuf, ones, zbuf):
    cid = lax.axis_index("c")
    sid = lax.axis_index("s")

    def _fill(j, _):
        zbuf[pl.ds(j * _NL, _NL)] = jnp.zeros((_NL,), jnp.float32)
        return 0
    lax.fori_loop(0, _ZB // _NL, _fill, 0)
    for j in range(_CH // _NL):
        ones[pl.ds(j * _NL, _NL)] = jnp.ones((_NL,), jnp.float32)

    for k in range(_M_STRIPE // _ZB):
        pltpu.sync_copy(zbuf, m_sh.at[pl.ds(sid * _M_STRIPE + k * _ZB, _ZB)])
    pltpu.sync_copy(zbuf.at[pl.ds(0, _OD_STRIPE)],
                    od_sh.at[pl.ds(sid * _OD_STRIPE, _OD_STRIPE)])
    plsc.subcore_barrier()

    for ch in range(_NCH):
        base = sid * _E_PER_W + ch * _CH
        pltpu.sync_copy(out_hbm.at[pl.ds(base, _CH)], obuf.at[ch])
        pltpu.sync_copy(in_hbm.at[pl.ds(base, _CH)], ibuf.at[ch])
        for g in range(_CH // _NL):
            o = obuf[ch, pl.ds(g * _NL, _NL)]
            i = ibuf[ch, pl.ds(g * _NL, _NL)]
            fbuf[ch, pl.ds(g * _NL, _NL)] = o * V + i
        pltpu.sync_copy(ones, m_sh.at[fbuf.at[ch]], add=True)
        pltpu.sync_copy(ones, od_sh.at[obuf.at[ch]], add=True)
    plsc.subcore_barrier()

    @pl.when(cid == 0)
    def _():
        for k in range(_M_STRIPE // _ZB):
            off = sid * _M_STRIPE + k * _ZB
            pltpu.sync_copy(m_sh.at[pl.ds(off, _ZB)], zbuf)
            pltpu.sync_copy(zbuf, m_out.at[pl.ds(off, _ZB)])
        pltpu.sync_copy(od_sh.at[pl.ds(sid * _OD_STRIPE, _OD_STRIPE)],
                        zbuf.at[pl.ds(0, _OD_STRIPE)])
        pltpu.sync_copy(zbuf.at[pl.ds(0, _OD_STRIPE)],
                        od_out.at[pl.ds(sid * _OD_STRIPE, _OD_STRIPE)])


@functools.lru_cache(maxsize=1)
def _get_sc_build():
    return pl.kernel(
        _sc_build_body,
        out_type=(jax.ShapeDtypeStruct((V * V,), jnp.float32),
                  jax.ShapeDtypeStruct((V,), jnp.float32)),
        mesh=plsc.VectorSubcoreMesh(core_axis_name="c", subcore_axis_name="s"),
        scratch_types=[
        pltpu.VMEM_SHARED((V * V,), jnp.float32),
        pltpu.VMEM_SHARED((V,), jnp.float32),
        pltpu.VMEM((_NCH, _CH), jnp.int32),
        pltpu.VMEM((_NCH, _CH), jnp.int32),
        pltpu.VMEM((_NCH, _CH), jnp.int32),
            pltpu.VMEM((_CH,), jnp.float32),
            pltpu.VMEM((_ZB,), jnp.float32),
        ],
    )


def _tc_body(x_ref, m_ref, od_ref, lenc_ref, lenr_ref, s_ref, e_ref,
             out_ref, logp, acc, res, sel):
    t = pl.program_id(0)
    xt = x_ref[0]
    colv = lax.broadcasted_iota(jnp.int32, (1, V), 1)

    @pl.when(t == 0)
    def _init():
        lc = lenc_ref[...]
        lr = lenr_ref[...]
        bi = lax.broadcasted_iota(jnp.int32, (B, B), 0)
        bj = lax.broadcasted_iota(jnp.int32, (B, B), 1)
        eq = (lc == lr) & (bj <= bi)
        src = jnp.sum(eq.astype(jnp.int32), axis=1, keepdims=True) - 1
        sel[...] = (bj == src).astype(jnp.float32)
        acc[...] = jnp.zeros((B, 1), jnp.float32)
        res[...] = jnp.zeros((B, 1), jnp.float32)
        smask = (colv == s_ref[0]) | (colv == s_ref[1]) \
            | (colv == s_ref[2]) | (colv == s_ref[3])
        logp[...] = jnp.where(smask, xt, LOG_EPS)

    @pl.when(t != 0)
    def _step():
        lp = logp[...]
        om = od_ref[...] > 0.0
        omf = om.astype(jnp.float32)
        m = jnp.max(jnp.where(om, lp, -1e30), axis=1, keepdims=True)
        ssum = jnp.sum(omf * jnp.exp(jnp.clip(lp - m, -200.0, 0.0)),
                       axis=1, keepdims=True)
        log_c = m + jnp.log(ssum)
        acc[...] += log_c
        q = omf * jnp.exp(jnp.clip(lp - log_c, LOG_EPS, 0.0))
        combined = jnp.dot(q, m_ref[...], preferred_element_type=jnp.float32)
        logp[...] = jnp.log(jnp.maximum(combined, EPS)) + xt

    lcur = logp[...]
    ecnt = ((colv == e_ref[0]).astype(jnp.float32)
            + (colv == e_ref[1]).astype(jnp.float32)
            + (colv == e_ref[2]).astype(jnp.float32)
            + (colv == e_ref[3]).astype(jnp.float32))
    me = jnp.max(jnp.where(ecnt > 0, lcur, -1e30), axis=1, keepdims=True)
    se = jnp.sum(ecnt * jnp.exp(jnp.clip(lcur - me, -200.0, 0.0)),
                 axis=1, keepdims=True)
    log_last = me + jnp.log(se) + acc[...]
    gathered = jnp.dot(sel[...], log_last, preferred_element_type=jnp.float32)
    res[...] = jnp.where(lenc_ref[...] == t + 1, gathered, res[...])
    out_ref[...] = -res[...]


def _tc_forward(xt, m_mat, odeg, lenc, lenr, start_idxs, end_idxs,
                interpret=False):
    return pl.pallas_call(
        _tc_body,
        grid=(T,),
        in_specs=[
            pl.BlockSpec((1, B, V), lambda t: (t, 0, 0)),
            pl.BlockSpec((V, V), lambda t: (0, 0)),
            pl.BlockSpec((1, V), lambda t: (0, 0)),
            pl.BlockSpec((B, 1), lambda t: (0, 0)),
            pl.BlockSpec((1, B), lambda t: (0, 0)),
            pl.BlockSpec(memory_space=pltpu.SMEM),
            pl.BlockSpec(memory_space=pltpu.SMEM),
        ],
        out_specs=pl.BlockSpec((B, 1), lambda t: (0, 0)),
        out_shape=jax.ShapeDtypeStruct((B, 1), jnp.float32),
        scratch_shapes=[
            pltpu.VMEM((B, V), jnp.float32),
            pltpu.VMEM((B, 1), jnp.float32),
            pltpu.VMEM((B, 1), jnp.float32),
            pltpu.VMEM((B, B), jnp.float32),
        ],
        compiler_params=pltpu.CompilerParams(
            dimension_semantics=("arbitrary",)),
        interpret=interpret,
    )(xt, m_mat, odeg, lenc, lenr, start_idxs, end_idxs)


def kernel(extracted_log_probs, target_lengths, in_idxs, out_idxs,
           start_idxs, end_idxs):
    xt = jnp.transpose(extracted_log_probs, (2, 1, 0))
    m_flat, odeg = _get_sc_build()(in_idxs, out_idxs)
    m_mat = m_flat.reshape(V, V)
    out = _tc_forward(xt, m_mat, odeg.reshape(1, V),
                      target_lengths.reshape(B, 1),
                      target_lengths.reshape(1, B),
                      start_idxs, end_idxs)
    return out.reshape(B)


# speedup vs baseline: 150.7517x; 1.0900x over previous
import functools
import math

import jax
import jax.numpy as jnp
from jax import lax
from jax.experimental import pallas as pl
from jax.experimental.pallas import tpu as pltpu
from jax.experimental.pallas import tpu_sc as plsc

LOG_EPS = -64.0
EPS = float(math.exp(-64.0))
V, B, T, E, S = 1024, 32, 64, 4096, 4

_NC, _NS, _NL = 2, 16, 16
_E_PER_W = E // _NS
_CH = 128
_NCH = _E_PER_W // _CH
_ZB = 8192
_M_STRIPE = (V * V) // _NS
_OD_STRIPE = V // _NS


def _sc_build_body(in_hbm, out_hbm, m_out, od_out, m_sh, od_sh,
                   obuf, ibuf, fbuf, ones, zbuf):
    cid = lax.axis_index("c")
    sid = lax.axis_index("s")

    def _fill(j, _):
        zbuf[pl.ds(j * _NL, _NL)] = jnp.zeros((_NL,), jnp.float32)
        return 0
    lax.fori_loop(0, _ZB // _NL, _fill, 0)
    for j in range(_CH // _NL):
        ones[pl.ds(j * _NL, _NL)] = jnp.ones((_NL,), jnp.float32)

    for k in range(_M_STRIPE // _ZB):
        pltpu.sync_copy(zbuf, m_sh.at[pl.ds(sid * _M_STRIPE + k * _ZB, _ZB)])
    pltpu.sync_copy(zbuf.at[pl.ds(0, _OD_STRIPE)],
                    od_sh.at[pl.ds(sid * _OD_STRIPE, _OD_STRIPE)])
    plsc.subcore_barrier()

    for ch in range(_NCH):
        base = sid * _E_PER_W + ch * _CH
        pltpu.sync_copy(out_hbm.at[pl.ds(base, _CH)], obuf.at[ch])
        pltpu.sync_copy(in_hbm.at[pl.ds(base, _CH)], ibuf.at[ch])
        for g in range(_CH // _NL):
            o = obuf[ch, pl.ds(g * _NL, _NL)]
            i = ibuf[ch, pl.ds(g * _NL, _NL)]
            fbuf[ch, pl.ds(g * _NL, _NL)] = o * V + i
        pltpu.sync_copy(ones, m_sh.at[fbuf.at[ch]], add=True)
        pltpu.sync_copy(ones, od_sh.at[obuf.at[ch]], add=True)
    plsc.subcore_barrier()

    @pl.when(cid == 0)
    def _():
        for k in range(_M_STRIPE // _ZB):
            off = sid * _M_STRIPE + k * _ZB
            pltpu.sync_copy(m_sh.at[pl.ds(off, _ZB)], zbuf)
            pltpu.sync_copy(zbuf, m_out.at[pl.ds(off, _ZB)])
        pltpu.sync_copy(od_sh.at[pl.ds(sid * _OD_STRIPE, _OD_STRIPE)],
                        zbuf.at[pl.ds(0, _OD_STRIPE)])
        pltpu.sync_copy(zbuf.at[pl.ds(0, _OD_STRIPE)],
                        od_out.at[pl.ds(sid * _OD_STRIPE, _OD_STRIPE)])


@functools.lru_cache(maxsize=1)
def _get_sc_build():
    return pl.kernel(
        _sc_build_body,
        out_type=(jax.ShapeDtypeStruct((V * V,), jnp.float32),
                  jax.ShapeDtypeStruct((V,), jnp.float32)),
        mesh=plsc.VectorSubcoreMesh(core_axis_name="c", subcore_axis_name="s"),
        scratch_types=[
        pltpu.VMEM_SHARED((V * V,), jnp.float32),
        pltpu.VMEM_SHARED((V,), jnp.float32),
        pltpu.VMEM((_NCH, _CH), jnp.int32),
        pltpu.VMEM((_NCH, _CH), jnp.int32),
        pltpu.VMEM((_NCH, _CH), jnp.int32),
            pltpu.VMEM((_CH,), jnp.float32),
            pltpu.VMEM((_ZB,), jnp.float32),
        ],
    )


def _tc_body(x_ref, m_ref, od_ref, lenc_ref, lenr_ref, s_ref, e_ref,
             out_ref, pst, acc, res, sel, ecnt):
    t = pl.program_id(0)
    xt = x_ref[0]

    @pl.when(t == 0)
    def _init():
        lc = lenc_ref[...]
        lr = lenr_ref[...]
        bi = lax.broadcasted_iota(jnp.int32, (B, B), 0)
        bj = lax.broadcasted_iota(jnp.int32, (B, B), 1)
        eq = (lc == lr) & (bj <= bi)
        src = jnp.sum(eq.astype(jnp.int32), axis=1, keepdims=True) - 1
        sel[...] = (bj == src).astype(jnp.float32)
        acc[...] = jnp.zeros((B, 1), jnp.float32)
        res[...] = jnp.zeros((B, 1), jnp.float32)
        colv = lax.broadcasted_iota(jnp.int32, (1, V), 1)
        smask = (colv == s_ref[0]) | (colv == s_ref[1]) \
            | (colv == s_ref[2]) | (colv == s_ref[3])
        pst[...] = jnp.where(smask, jnp.exp(xt), EPS)
        ecnt[...] = ((colv == e_ref[0]).astype(jnp.float32)
                     + (colv == e_ref[1]).astype(jnp.float32)
                     + (colv == e_ref[2]).astype(jnp.float32)
                     + (colv == e_ref[3]).astype(jnp.float32))

    @pl.when(t != 0)
    def _step():
        p = pst[...]
        e = od_ref[...] * p
        s = jnp.sum(e, axis=1, keepdims=True)
        seps = s * EPS
        ef = jnp.maximum(e, seps).astype(jnp.bfloat16)
        a = jnp.dot(ef, m_ref[...], preferred_element_type=jnp.float32)
        acc[...] += jnp.log(s)
        pst[...] = jnp.maximum(a, seps) * jnp.exp(xt) * (1.0 / s)

    s4 = jnp.sum(pst[...] * ecnt[...], axis=1, keepdims=True)
    log_last = jnp.log(s4) + acc[...]
    gathered = jnp.dot(sel[...], log_last, preferred_element_type=jnp.float32)
    res[...] = jnp.where(lenc_ref[...] == t + 1, gathered, res[...])
    out_ref[...] = -res[...]


def _tc_forward(xt, m_mat, odeg, lenc, lenr, start_idxs, end_idxs,
                interpret=False):
    return pl.pallas_call(
        _tc_body,
        grid=(T,),
        in_specs=[
            pl.BlockSpec((1, B, V), lambda t: (t, 0, 0)),
            pl.BlockSpec((V, V), lambda t: (0, 0)),
            pl.BlockSpec((1, V), lambda t: (0, 0)),
            pl.BlockSpec((B, 1), lambda t: (0, 0)),
            pl.BlockSpec((1, B), lambda t: (0, 0)),
            pl.BlockSpec(memory_space=pltpu.SMEM),
            pl.BlockSpec(memory_space=pltpu.SMEM),
        ],
        out_specs=pl.BlockSpec((B, 1), lambda t: (0, 0)),
        out_shape=jax.ShapeDtypeStruct((B, 1), jnp.float32),
        scratch_shapes=[
            pltpu.VMEM((B, V), jnp.float32),
            pltpu.VMEM((B, 1), jnp.float32),
            pltpu.VMEM((B, 1), jnp.float32),
            pltpu.VMEM((B, B), jnp.float32),
            pltpu.VMEM((1, V), jnp.float32),
        ],
        compiler_params=pltpu.CompilerParams(
            dimension_semantics=("arbitrary",)),
        interpret=interpret,
    )(xt, m_mat, odeg, lenc, lenr, start_idxs, end_idxs)


def kernel(extracted_log_probs, target_lengths, in_idxs, out_idxs,
           start_idxs, end_idxs):
    xt = jnp.transpose(extracted_log_probs, (2, 1, 0))
    m_flat, odeg = _get_sc_build()(in_idxs, out_idxs)
    m_mat = m_flat.reshape(V, V).astype(jnp.bfloat16)
    omask = (odeg > 0).astype(jnp.float32).reshape(1, V)
    out = _tc_forward(xt, m_mat, omask,
                      target_lengths.reshape(B, 1),
                      target_lengths.reshape(1, B),
                      start_idxs, end_idxs)
    return out.reshape(B)


# speedup vs baseline: 176.3524x; 1.1698x over previous
import functools
import math

import jax
import jax.numpy as jnp
from jax import lax
from jax.experimental import pallas as pl
from jax.experimental.pallas import tpu as pltpu
from jax.experimental.pallas import tpu_sc as plsc

LOG_EPS = -64.0
EPS = float(math.exp(-64.0))
V, B, T, E, S = 1024, 32, 64, 4096, 4

_NC, _NS, _NL = 2, 16, 16
_E_PER_W = E // _NS
_CH = 128
_NCH = _E_PER_W // _CH
_ZB = 8192
_M_STRIPE = (V * V) // _NS
_OD_STRIPE = V // _NS


def _sc_build_body(in_hbm, out_hbm, m_out, od_out, m_sh, od_sh,
                   obuf, ibuf, fbuf, ones, zbuf):
    cid = lax.axis_index("c")
    sid = lax.axis_index("s")

    def _fill(j, _):
        zbuf[pl.ds(j * _NL, _NL)] = jnp.zeros((_NL,), jnp.float32)
        return 0
    lax.fori_loop(0, _ZB // _NL, _fill, 0)
    for j in range(_CH // _NL):
        ones[pl.ds(j * _NL, _NL)] = jnp.ones((_NL,), jnp.float32)

    for k in range(_M_STRIPE // _ZB):
        pltpu.sync_copy(zbuf, m_sh.at[pl.ds(sid * _M_STRIPE + k * _ZB, _ZB)])
    pltpu.sync_copy(zbuf.at[pl.ds(0, _OD_STRIPE)],
                    od_sh.at[pl.ds(sid * _OD_STRIPE, _OD_STRIPE)])
    plsc.subcore_barrier()

    for ch in range(_NCH):
        base = sid * _E_PER_W + ch * _CH
        pltpu.sync_copy(out_hbm.at[pl.ds(base, _CH)], obuf.at[ch])
        pltpu.sync_copy(in_hbm.at[pl.ds(base, _CH)], ibuf.at[ch])
        for g in range(_CH // _NL):
            o = obuf[ch, pl.ds(g * _NL, _NL)]
            i = ibuf[ch, pl.ds(g * _NL, _NL)]
            fbuf[ch, pl.ds(g * _NL, _NL)] = o * V + i
        pltpu.sync_copy(ones, m_sh.at[fbuf.at[ch]], add=True)
        pltpu.sync_copy(ones, od_sh.at[obuf.at[ch]], add=True)
    plsc.subcore_barrier()

    @pl.when(cid == 0)
    def _():
        for k in range(_M_STRIPE // _ZB):
            off = sid * _M_STRIPE + k * _ZB
            pltpu.sync_copy(m_sh.at[pl.ds(off, _ZB)], zbuf)
            pltpu.sync_copy(zbuf, m_out.at[pl.ds(off, _ZB)])
        pltpu.sync_copy(od_sh.at[pl.ds(sid * _OD_STRIPE, _OD_STRIPE)],
                        zbuf.at[pl.ds(0, _OD_STRIPE)])
        pltpu.sync_copy(zbuf.at[pl.ds(0, _OD_STRIPE)],
                        od_out.at[pl.ds(sid * _OD_STRIPE, _OD_STRIPE)])


@functools.lru_cache(maxsize=1)
def _get_sc_build():
    return pl.kernel(
        _sc_build_body,
        out_type=(jax.ShapeDtypeStruct((V * V,), jnp.float32),
                  jax.ShapeDtypeStruct((V,), jnp.float32)),
        mesh=plsc.VectorSubcoreMesh(core_axis_name="c", subcore_axis_name="s"),
        scratch_types=[
        pltpu.VMEM_SHARED((V * V,), jnp.float32),
        pltpu.VMEM_SHARED((V,), jnp.float32),
        pltpu.VMEM((_NCH, _CH), jnp.int32),
        pltpu.VMEM((_NCH, _CH), jnp.int32),
        pltpu.VMEM((_NCH, _CH), jnp.int32),
            pltpu.VMEM((_CH,), jnp.float32),
            pltpu.VMEM((_ZB,), jnp.float32),
        ],
    )


def _tc_body(x_ref, m_ref, od_ref, lenc_ref, lenr_ref, s_ref, e_ref,
             out_ref):
    colv = lax.broadcasted_iota(jnp.int32, (1, V), 1)
    smask = (colv == s_ref[0]) | (colv == s_ref[1]) \
        | (colv == s_ref[2]) | (colv == s_ref[3])
    ecnt = ((colv == e_ref[0]).astype(jnp.float32)
            + (colv == e_ref[1]).astype(jnp.float32)
            + (colv == e_ref[2]).astype(jnp.float32)
            + (colv == e_ref[3]).astype(jnp.float32))
    lc = lenc_ref[...]
    lr = lenr_ref[...]
    bi = lax.broadcasted_iota(jnp.int32, (B, B), 0)
    bj = lax.broadcasted_iota(jnp.int32, (B, B), 1)
    eq = (lc == lr) & (bj <= bi)
    src = jnp.sum(eq.astype(jnp.int32), axis=1, keepdims=True) - 1
    sel = (bj == src).astype(jnp.float32)
    om = od_ref[...]

    pst = jnp.where(smask, jnp.exp(x_ref[0]), EPS)
    acc = jnp.zeros((B, 1), jnp.float32)
    res = jnp.zeros((B, 1), jnp.float32)
    for t in range(T):
        if t > 0:
            e = om * pst
            s = jnp.sum(e, axis=1, keepdims=True)
            seps = s * EPS
            ef = jnp.maximum(e, seps).astype(jnp.bfloat16)
            a = jnp.dot(ef, m_ref[...],
                        preferred_element_type=jnp.float32)
            acc = acc + jnp.log(s)
            pst = jnp.maximum(a, seps) * jnp.exp(x_ref[t]) * (1.0 / s)
        s4 = jnp.sum(pst * ecnt, axis=1, keepdims=True)
        log_last = jnp.log(s4) + acc
        gathered = jax.lax.dot_general(
            sel, log_last, (((1,), (0,)), ((), ())),
            precision=jax.lax.Precision.HIGHEST,
            preferred_element_type=jnp.float32)
        res = jnp.where(lc == t + 1, gathered, res)
    out_ref[...] = -res


def _tc_forward(xt, m_mat, odeg, lenc, lenr, start_idxs, end_idxs,
                interpret=False):
    return pl.pallas_call(
        _tc_body,
        in_specs=[
            pl.BlockSpec((T, B, V), lambda: (0, 0, 0)),
            pl.BlockSpec((V, V), lambda: (0, 0)),
            pl.BlockSpec((1, V), lambda: (0, 0)),
            pl.BlockSpec((B, 1), lambda: (0, 0)),
            pl.BlockSpec((1, B), lambda: (0, 0)),
            pl.BlockSpec(memory_space=pltpu.SMEM),
            pl.BlockSpec(memory_space=pltpu.SMEM),
        ],
        out_specs=pl.BlockSpec((B, 1), lambda: (0, 0)),
        out_shape=jax.ShapeDtypeStruct((B, 1), jnp.float32),
        interpret=interpret,
    )(xt, m_mat, odeg, lenc, lenr, start_idxs, end_idxs)


def kernel(extracted_log_probs, target_lengths, in_idxs, out_idxs,
           start_idxs, end_idxs):
    xt = jnp.transpose(extracted_log_probs, (2, 1, 0))
    m_flat, odeg = _get_sc_build()(in_idxs, out_idxs)
    m_mat = m_flat.reshape(V, V).astype(jnp.bfloat16)
    omask = (odeg > 0).astype(jnp.float32).reshape(1, V)
    out = _tc_forward(xt, m_mat, omask,
                      target_lengths.reshape(B, 1),
                      target_lengths.reshape(1, B),
                      start_idxs, end_idxs)
    return out.reshape(B)


# speedup vs baseline: 178.9760x; 1.0149x over previous
import functools
import math

import jax
import jax.numpy as jnp
from jax import lax
from jax.experimental import pallas as pl
from jax.experimental.pallas import tpu as pltpu
from jax.experimental.pallas import tpu_sc as plsc

LOG_EPS = -64.0
EPS = float(math.exp(-64.0))
V, B, T, E, S = 1024, 32, 64, 4096, 4

_NC, _NS, _NL = 2, 16, 16
_E_PER_W = E // _NS
_CH = 128
_NCH = _E_PER_W // _CH
_ZB = 8192
_M_STRIPE = (V * V) // _NS
_OD_STRIPE = V // _NS
_NK = _M_STRIPE // _ZB


def _sc_build_body(in_hbm, out_hbm, zeros_hbm, ones_hbm, m_out, od_out,
                   m_sh, od_sh, obuf, ibuf, fbuf, ones, odbuf, stage, sem):
    cid = lax.axis_index("c")
    sid = lax.axis_index("s")

    @pl.when(cid == 0)
    def _build():
        pltpu.sync_copy(ones_hbm, ones)

        h = pltpu.async_copy(
            zeros_hbm, m_sh.at[pl.ds(sid * _M_STRIPE, _M_STRIPE)], sem)
        @pl.when(sid == 0)
        def _():
            pltpu.sync_copy(zeros_hbm.at[pl.ds(0, V)], od_sh)
        h.wait()
        plsc.subcore_barrier()

        for ch in range(_NCH):
            base = sid * _E_PER_W + ch * _CH
            pltpu.sync_copy(out_hbm.at[pl.ds(base, _CH)], obuf.at[ch])
            pltpu.sync_copy(in_hbm.at[pl.ds(base, _CH)], ibuf.at[ch])
            for g in range(_CH // _NL):
                o = obuf[ch, pl.ds(g * _NL, _NL)]
                i = ibuf[ch, pl.ds(g * _NL, _NL)]
                fbuf[ch, pl.ds(g * _NL, _NL)] = o * V + i
            pltpu.sync_copy(ones, m_sh.at[fbuf.at[ch]], add=True)
            pltpu.sync_copy(ones, od_sh.at[obuf.at[ch]], add=True)
        plsc.subcore_barrier()

        handles = []
        for k in range(_NK):
            off = sid * _M_STRIPE + k * _ZB
            slot = (k % 2) * _ZB
            if k >= 2:
                handles[k - 2].wait()
            pltpu.sync_copy(m_sh.at[pl.ds(off, _ZB)],
                            stage.at[pl.ds(slot, _ZB)])
            handles.append(
                pltpu.async_copy(stage.at[pl.ds(slot, _ZB)],
                                 m_out.at[pl.ds(off, _ZB)], sem))
        @pl.when(sid == 0)
        def _():
            pltpu.sync_copy(od_sh, odbuf)
            pltpu.sync_copy(odbuf, od_out)
        handles[_NK - 2].wait()
        handles[_NK - 1].wait()


@functools.lru_cache(maxsize=1)
def _get_sc_build():
    return pl.kernel(
        _sc_build_body,
        out_type=(jax.ShapeDtypeStruct((V * V,), jnp.float32),
                  jax.ShapeDtypeStruct((V,), jnp.float32)),
        mesh=plsc.VectorSubcoreMesh(core_axis_name="c", subcore_axis_name="s"),
        scratch_types=[
            pltpu.VMEM_SHARED((V * V,), jnp.float32),
            pltpu.VMEM_SHARED((V,), jnp.float32),
            pltpu.VMEM((_NCH, _CH), jnp.int32),
            pltpu.VMEM((_NCH, _CH), jnp.int32),
            pltpu.VMEM((_NCH, _CH), jnp.int32),
            pltpu.VMEM((_CH,), jnp.float32),
            pltpu.VMEM((V,), jnp.float32),
            pltpu.VMEM((2 * _ZB,), jnp.float32),
            pltpu.SemaphoreType.DMA,
        ],
    )


def _tc_body(x_ref, m_ref, od_ref, lenc_ref, lenr_ref, s_ref, e_ref,
             out_ref):
    colv = lax.broadcasted_iota(jnp.int32, (1, V), 1)
    smask = (colv == s_ref[0]) | (colv == s_ref[1]) \
        | (colv == s_ref[2]) | (colv == s_ref[3])
    ecnt = ((colv == e_ref[0]).astype(jnp.float32)
            + (colv == e_ref[1]).astype(jnp.float32)
            + (colv == e_ref[2]).astype(jnp.float32)
            + (colv == e_ref[3]).astype(jnp.float32))
    lc = lenc_ref[...]
    lr = lenr_ref[...]
    bi = lax.broadcasted_iota(jnp.int32, (B, B), 0)
    bj = lax.broadcasted_iota(jnp.int32, (B, B), 1)
    eq = (lc == lr) & (bj <= bi)
    src = jnp.sum(eq.astype(jnp.int32), axis=1, keepdims=True) - 1
    sel = (bj == src).astype(jnp.float32)
    om = od_ref[...]

    pst = jnp.where(smask, jnp.exp(x_ref[0]), EPS)
    acc = jnp.zeros((B, 1), jnp.float32)
    res = jnp.zeros((B, 1), jnp.float32)
    for t in range(T):
        if t > 0:
            e = om * pst
            s = jnp.sum(e, axis=1, keepdims=True)
            seps = s * EPS
            ef = jnp.maximum(e, seps).astype(jnp.bfloat16)
            a = jnp.dot(ef, m_ref[...],
                        preferred_element_type=jnp.float32)
            acc = acc + jnp.log(s)
            pst = jnp.maximum(a, seps) * jnp.exp(x_ref[t]) * (1.0 / s)
        s4 = jnp.sum(pst * ecnt, axis=1, keepdims=True)
        log_last = jnp.log(s4) + acc
        gathered = jax.lax.dot_general(
            sel, log_last, (((1,), (0,)), ((), ())),
            precision=jax.lax.Precision.HIGHEST,
            preferred_element_type=jnp.float32)
        res = jnp.where(lc == t + 1, gathered, res)
    out_ref[...] = -res


def _tc_forward(xt, m_mat, odeg, lenc, lenr, start_idxs, end_idxs,
                interpret=False):
    return pl.pallas_call(
        _tc_body,
        in_specs=[
            pl.BlockSpec((T, B, V), lambda: (0, 0, 0)),
            pl.BlockSpec((V, V), lambda: (0, 0)),
            pl.BlockSpec((1, V), lambda: (0, 0)),
            pl.BlockSpec((B, 1), lambda: (0, 0)),
            pl.BlockSpec((1, B), lambda: (0, 0)),
            pl.BlockSpec(memory_space=pltpu.SMEM),
            pl.BlockSpec(memory_space=pltpu.SMEM),
        ],
        out_specs=pl.BlockSpec((B, 1), lambda: (0, 0)),
        out_shape=jax.ShapeDtypeStruct((B, 1), jnp.float32),
        interpret=interpret,
    )(xt, m_mat, odeg, lenc, lenr, start_idxs, end_idxs)


def kernel(extracted_log_probs, target_lengths, in_idxs, out_idxs,
           start_idxs, end_idxs):
    xt = jnp.transpose(extracted_log_probs, (2, 1, 0))
    m_flat, odeg = _get_sc_build()(
        in_idxs, out_idxs,
        jnp.zeros((_M_STRIPE,), jnp.float32),
        jnp.ones((_CH,), jnp.float32))
    out = _tc_forward(xt, m_flat.reshape(V, V).astype(jnp.bfloat16),
                      (odeg > 0).astype(jnp.float32).reshape(1, V),
                      target_lengths.reshape(B, 1),
                      target_lengths.reshape(1, B),
                      start_idxs, end_idxs)
    return out.reshape(B)
